# baseline (speedup 1.0000x reference)
import functools

import jax
import jax.numpy as jnp
from jax import lax
from jax.experimental import pallas as pl
from jax.experimental.pallas import tpu as pltpu
from jax.experimental.pallas import tpu_sc as plsc

N = 10000
E = 320000
H = 128
HH = 64
NC = 2
NS = 16
EPT = E // NS
KB = 80
NBLK = EPT // KB
NWB = 624
EPS = 1e-6
EPS_BN = 1e-5

_mesh = plsc.VectorSubcoreMesh(core_axis_name="c", subcore_axis_name="s")


def _edge_body(write_raw, ce, dh, eh, bh, srcr, dstr, zz, *refs):
  if write_raw:
    raw_o, nd_o = refs[0], refs[1]
    scratch = refs[2:]
  else:
    nd_o = refs[0]
    scratch = refs[1:]
  (isr, idr, ig, ih, ce_v, dh_v, eh_v, bh_v, raw_v, cs_v, acc,
   s0, s1, s2, s3) = scratch

  c = lax.axis_index("c")
  s = lax.axis_index("s")
  cN = c * N
  cE = c * E

  pltpu.sync_copy(zz.at[pl.ds(0, NWB)], acc.at[pl.ds(s * NWB, NWB)])

  @pl.when(s == NS - 1)
  def _():
    pltpu.sync_copy(zz.at[pl.ds(0, 16)], acc.at[pl.ds(NS * NWB, 16)])

  plsc.subcore_barrier()

  base0 = s * EPT

  def blk(b, carry):
    base = base0 + b * KB
    pltpu.sync_copy(srcr.at[pl.ds(base, KB)], isr)
    pltpu.sync_copy(dstr.at[pl.ds(base, KB)], idr)

    def off(i, _):
      sl = pl.ds(i * 16, 16)
      ig[sl] = isr[sl] + cN
      ih[sl] = idr[sl] + cN
      return 0

    lax.fori_loop(0, KB // 16, off, 0)

    cp0 = pltpu.async_copy(ce.at[pl.ds(cE + base, KB)], ce_v, s0)
    cp1 = pltpu.async_copy(dh.at[ig], dh_v, s1)
    cp2 = pltpu.async_copy(eh.at[ih], eh_v, s2)
    cp3 = pltpu.async_copy(bh.at[ig], bh_v, s3)
    cp0.wait()
    cp1.wait()
    cp2.wait()
    cp3.wait()

    def row(r, _):
      for j in range(4):
        sl = pl.ds(j * 16, 16)
        rv = ce_v[r, sl] + dh_v[r, sl] + eh_v[r, sl]
        raw_v[r, sl] = rv
        sg = 1.0 / (1.0 + jnp.exp(-rv))
        cs_v[r, pl.ds(HH + j * 16, 16)] = sg
        cs_v[r, sl] = sg * bh_v[r, sl]
      return 0

    lax.fori_loop(0, KB, row, 0)

    if write_raw:
      pltpu.sync_copy(raw_v, raw_o.at[pl.ds(cE + base, KB)])
    pltpu.sync_copy(cs_v, acc.at[idr], add=True)
    return carry

  lax.fori_loop(0, NBLK, blk, 0)

  plsc.subcore_barrier()

  pltpu.sync_copy(acc.at[pl.ds(s * NWB, NWB)],
                  nd_o.at[pl.ds(cN + s * NWB, NWB)])

  @pl.when(s == NS - 1)
  def _():
    pltpu.sync_copy(acc.at[pl.ds(NS * NWB, 16)],
                    nd_o.at[pl.ds(cN + NS * NWB, 16)])


def _make_edge_kernel(write_raw):
  out_type = [jax.ShapeDtypeStruct((NC * N, H), jnp.float32)]
  if write_raw:
    out_type = [jax.ShapeDtypeStruct((NC * E, HH), jnp.float32)] + out_type
  scratch = [
      pltpu.VMEM((KB,), jnp.int32),
      pltpu.VMEM((KB,), jnp.int32),
      pltpu.VMEM((KB,), jnp.int32),
      pltpu.VMEM((KB,), jnp.int32),
      pltpu.VMEM((KB, HH), jnp.float32),
      pltpu.VMEM((KB, HH), jnp.float32),
      pltpu.VMEM((KB, HH), jnp.float32),
      pltpu.VMEM((KB, HH), jnp.float32),
      pltpu.VMEM((KB, HH), jnp.float32),
      pltpu.VMEM((KB, H), jnp.float32),
      pltpu.VMEM_SHARED((N, H), jnp.float32),
      pltpu.SemaphoreType.DMA,
      pltpu.SemaphoreType.DMA,
      pltpu.SemaphoreType.DMA,
      pltpu.SemaphoreType.DMA,
  ]
  return pl.kernel(
      functools.partial(_edge_body, write_raw),
      out_type=out_type,
      mesh=_mesh,
      scratch_types=scratch,
      compiler_params=pltpu.CompilerParams(use_tc_tiling_on_sc=False),
  )


_edge_mid = _make_edge_kernel(True)
_edge_last = _make_edge_kernel(False)


def _bn_relu(x, g, b):
  mean = jnp.mean(x, axis=0, keepdims=True)
  var = jnp.var(x, axis=0, keepdims=True)
  return jax.nn.relu((x - mean) / jnp.sqrt(var + EPS_BN) * g + b)


def _split2(x):
  return jnp.concatenate([x[:, :HH], x[:, HH:]], axis=0)


def kernel(box_feats, text_feats, edge_feat, edge_index, params):
  p = params
  src = edge_index[0].astype(jnp.int32)
  dst = edge_index[1].astype(jnp.int32)
  zz = jnp.zeros((NWB, H), jnp.float32)

  h = jnp.concatenate([box_feats, text_feats], axis=1) @ p['node_enc_w'] + p['node_enc_b']
  e = edge_feat @ p['edge_enc_w'] + p['edge_enc_b']

  n_layers = len(p['layers'])
  for li, lp in enumerate(p['layers']):
    last = li == n_layers - 1
    h_in, e_in = h, e
    Ah = h @ lp['Aw'] + lp['Ab']
    Bh = h @ lp['Bw'] + lp['Bb']
    Ce = e @ lp['Cw'] + lp['Cb']
    Dh = h @ lp['Dw'] + lp['Db']
    Eh = h @ lp['Ew'] + lp['Eb']

    ce2 = _split2(Ce)
    dh2 = _split2(Dh)
    eh2 = _split2(Eh)
    bh2 = _split2(Bh)

    if last:
      (nd,) = _edge_last(ce2, dh2, eh2, bh2, src, dst, zz)
    else:
      raw2, nd = _edge_mid(ce2, dh2, eh2, bh2, src, dst, zz)

    num = jnp.concatenate([nd[:N, :HH], nd[N:, :HH]], axis=1)
    den = jnp.concatenate([nd[:N, HH:], nd[N:, HH:]], axis=1)

    h_new = Ah + num / (den + EPS)
    h = h_in + _bn_relu(h_new, lp['bnh_g'], lp['bnh_b'])
    if not last:
      raw = jnp.concatenate([raw2[:E], raw2[E:]], axis=1)
      e = e_in + _bn_relu(raw, lp['bne_g'], lp['bne_b'])

  m = jax.nn.relu(h @ p['mlp1_w'] + p['mlp1_b'])
  return m @ p['mlp2_w'] + p['mlp2_b']


# speedup vs baseline: 1.1346x; 1.1346x over previous
import functools

import jax
import jax.numpy as jnp
from jax import lax
from jax.experimental import pallas as pl
from jax.experimental.pallas import tpu as pltpu
from jax.experimental.pallas import tpu_sc as plsc

N = 10000
E = 320000
H = 128
HH = 64
NC = 2
NS = 16
EPT = E // NS
KB = 16
NBLK = EPT // KB
NWB = 624
EPS = 1e-6
EPS_BN = 1e-5

_mesh = plsc.VectorSubcoreMesh(core_axis_name="c", subcore_axis_name="s")


class _BufSet:
  def __init__(self, is_v, id_v, ce_v, db_v, eh_v, cs_v,
               s_idx, s_ce, s_db, s_eh, s_raw, s_sc):
    self.is_v = is_v
    self.id_v = id_v
    self.ce_v = ce_v
    self.db_v = db_v
    self.eh_v = eh_v
    self.cs_v = cs_v
    self.s_idx = s_idx
    self.s_ce = s_ce
    self.s_db = s_db
    self.s_eh = s_eh
    self.s_raw = s_raw
    self.s_sc = s_sc


def _edge_body(write_raw, ce, db, eh, srcr, dstr, zz, *refs):
  if write_raw:
    raw_o, nd_o = refs[0], refs[1]
    scratch = refs[2:]
  else:
    raw_o = None
    nd_o = refs[0]
    scratch = refs[1:]
  acc = scratch[0]
  sets = [_BufSet(*scratch[1 + i * 12:1 + (i + 1) * 12]) for i in range(2)]

  c = lax.axis_index("c")
  s = lax.axis_index("s")
  cN = c * N
  cE = c * E
  dummy = lax.iota(jnp.int32, 16)

  pltpu.sync_copy(zz.at[pl.ds(0, NWB)], acc.at[pl.ds(s * NWB, NWB)])

  @pl.when(s == NS - 1)
  def _():
    pltpu.sync_copy(zz.at[pl.ds(0, 16)], acc.at[pl.ds(NS * NWB, 16)])

  base0 = s * EPT

  def issue_idx(b, S):
    o = base0 + b * KB
    pltpu.async_copy(srcr.at[pl.ds(o, KB)], S.is_v, S.s_idx)
    pltpu.async_copy(dstr.at[pl.ds(o, KB)], S.id_v, S.s_idx)

  def issue(b, S):
    o = base0 + b * KB
    @pl.when(b >= 2)
    def _():
      if write_raw:
        pltpu.make_async_copy(S.ce_v, raw_o.at[pl.ds(cE, KB)], S.s_raw).wait()

    pltpu.make_async_copy(srcr.at[pl.ds(base0, KB)], S.is_v, S.s_idx).wait()
    pltpu.make_async_copy(dstr.at[pl.ds(base0, KB)], S.id_v, S.s_idx).wait()

    sv = S.is_v[...]
    dv = S.id_v[...]
    pltpu.async_copy(ce.at[pl.ds(cE + o, KB)], S.ce_v, S.s_ce)
    pltpu.async_copy(db.at[sv + cN], S.db_v, S.s_db)
    pltpu.async_copy(eh.at[dv + cN], S.eh_v, S.s_eh)

    @pl.when(b + 2 < NBLK)
    def _():
      issue_idx(b + 2, S)

    return dv

  def process(b, S, dv):
    o = base0 + b * KB
    pltpu.make_async_copy(ce.at[pl.ds(cE + o, KB)], S.ce_v, S.s_ce).wait()
    pltpu.make_async_copy(db.at[dummy], S.db_v, S.s_db).wait()
    pltpu.make_async_copy(eh.at[dummy], S.eh_v, S.s_eh).wait()

    def row(r, _):
      for rr in range(2):
        rw = 2 * r + rr
        for j in range(4):
          sl = pl.ds(j * 16, 16)
          sh = pl.ds(HH + j * 16, 16)
          rv = S.ce_v[rw, sl] + S.db_v[rw, sl] + S.eh_v[rw, sl]
          S.ce_v[rw, sl] = rv
          sg = 1.0 / (1.0 + jnp.exp(-rv))
          S.cs_v[rw, sh] = sg
          S.cs_v[rw, sl] = sg * S.db_v[rw, sh]
      return 0

    lax.fori_loop(0, KB // 2, row, 0)

    if write_raw:
      pltpu.async_copy(S.ce_v, raw_o.at[pl.ds(cE + o, KB)], S.s_raw)
    pltpu.async_copy(S.cs_v, acc.at[dv], S.s_sc, add=True).wait()

  issue_idx(0, sets[0])
  issue_idx(1, sets[1])
  plsc.subcore_barrier()
  dvA = issue(0, sets[0])
  dvB = issue(1, sets[1])

  def g_body(g, carry):
    dvA, dvB = carry
    b0 = 2 * g
    process(b0, sets[0], dvA)
    dvA = issue(b0 + 2, sets[0])
    process(b0 + 1, sets[1], dvB)
    dvB = issue(b0 + 3, sets[1])
    return (dvA, dvB)

  dvA, dvB = lax.fori_loop(0, NBLK // 2 - 1, g_body, (dvA, dvB))
  process(NBLK - 2, sets[0], dvA)
  process(NBLK - 1, sets[1], dvB)

  if write_raw:
    for S in sets:
      pltpu.make_async_copy(S.ce_v, raw_o.at[pl.ds(cE, KB)], S.s_raw).wait()

  plsc.subcore_barrier()

  pltpu.sync_copy(acc.at[pl.ds(s * NWB, NWB)],
                  nd_o.at[pl.ds(cN + s * NWB, NWB)])

  @pl.when(s == NS - 1)
  def _():
    pltpu.sync_copy(acc.at[pl.ds(NS * NWB, 16)],
                    nd_o.at[pl.ds(cN + NS * NWB, 16)])


def _make_edge_kernel(write_raw):
  out_type = [jax.ShapeDtypeStruct((NC * N, H), jnp.float32)]
  if write_raw:
    out_type = [jax.ShapeDtypeStruct((NC * E, HH), jnp.float32)] + out_type
  one_set = [
      pltpu.VMEM((KB,), jnp.int32),
      pltpu.VMEM((KB,), jnp.int32),
      pltpu.VMEM((KB, HH), jnp.float32),
      pltpu.VMEM((KB, H), jnp.float32),
      pltpu.VMEM((KB, HH), jnp.float32),
      pltpu.VMEM((KB, H), jnp.float32),
      pltpu.SemaphoreType.DMA,
      pltpu.SemaphoreType.DMA,
      pltpu.SemaphoreType.DMA,
      pltpu.SemaphoreType.DMA,
      pltpu.SemaphoreType.DMA,
      pltpu.SemaphoreType.DMA,
  ]
  scratch = [
      pltpu.VMEM_SHARED((N, H), jnp.float32),
  ] + one_set + one_set
  return pl.kernel(
      functools.partial(_edge_body, write_raw),
      out_type=out_type,
      mesh=_mesh,
      scratch_types=scratch,
      compiler_params=pltpu.CompilerParams(use_tc_tiling_on_sc=False),
  )


_edge_mid = _make_edge_kernel(True)
_edge_last = _make_edge_kernel(False)


def _bn_relu(x, g, b):
  mean = jnp.mean(x, axis=0, keepdims=True)
  var = jnp.var(x, axis=0, keepdims=True)
  return jax.nn.relu((x - mean) / jnp.sqrt(var + EPS_BN) * g + b)


def _split2(x):
  return jnp.concatenate([x[:, :HH], x[:, HH:]], axis=0)


def kernel(box_feats, text_feats, edge_feat, edge_index, params):
  p = params
  src = edge_index[0].astype(jnp.int32)
  dst = edge_index[1].astype(jnp.int32)
  zz = jnp.zeros((NWB, H), jnp.float32)

  h = jnp.concatenate([box_feats, text_feats], axis=1) @ p['node_enc_w'] + p['node_enc_b']
  e = edge_feat @ p['edge_enc_w'] + p['edge_enc_b']

  n_layers = len(p['layers'])
  for li, lp in enumerate(p['layers']):
    last = li == n_layers - 1
    h_in, e_in = h, e
    Ah = h @ lp['Aw'] + lp['Ab']
    Bh = h @ lp['Bw'] + lp['Bb']
    Ce = e @ lp['Cw'] + lp['Cb']
    Dh = h @ lp['Dw'] + lp['Db']
    Eh = h @ lp['Ew'] + lp['Eb']

    ce2 = _split2(Ce)
    eh2 = _split2(Eh)
    db2 = jnp.concatenate([
        jnp.concatenate([Dh[:, :HH], Bh[:, :HH]], axis=1),
        jnp.concatenate([Dh[:, HH:], Bh[:, HH:]], axis=1),
    ], axis=0)

    if last:
      (nd,) = _edge_last(ce2, db2, eh2, src, dst, zz)
    else:
      raw2, nd = _edge_mid(ce2, db2, eh2, src, dst, zz)

    num = jnp.concatenate([nd[:N, :HH], nd[N:, :HH]], axis=1)
    den = jnp.concatenate([nd[:N, HH:], nd[N:, HH:]], axis=1)

    h_new = Ah + num / (den + EPS)
    h = h_in + _bn_relu(h_new, lp['bnh_g'], lp['bnh_b'])
    if not last:
      raw = jnp.concatenate([raw2[:E], raw2[E:]], axis=1)
      e = e_in + _bn_relu(raw, lp['bne_g'], lp['bne_b'])

  m = jax.nn.relu(h @ p['mlp1_w'] + p['mlp1_b'])
  return m @ p['mlp2_w'] + p['mlp2_b']


# speedup vs baseline: 1.1481x; 1.0118x over previous
import functools

import jax
import jax.numpy as jnp
from jax import lax
from jax.experimental import pallas as pl
from jax.experimental.pallas import tpu as pltpu
from jax.experimental.pallas import tpu_sc as plsc

N = 10000
E = 320000
H = 128
HH = 64
NC = 2
NS = 16
EPT = E // NS
KB = 32
NBLK = EPT // KB
NWB = 624
EPS = 1e-6
EPS_BN = 1e-5

_mesh = plsc.VectorSubcoreMesh(core_axis_name="c", subcore_axis_name="s")


class _BufSet:
  def __init__(self, is_raw, id_raw, ig, ih, idr, ce_v, db_v, eh_v, cs_v,
               s_idx, s_ce, s_db, s_eh, s_raw, s_sc):
    self.is_raw = is_raw
    self.id_raw = id_raw
    self.ig = ig
    self.ih = ih
    self.idr = idr
    self.ce_v = ce_v
    self.db_v = db_v
    self.eh_v = eh_v
    self.cs_v = cs_v
    self.s_idx = s_idx
    self.s_ce = s_ce
    self.s_db = s_db
    self.s_eh = s_eh
    self.s_raw = s_raw
    self.s_sc = s_sc


def _edge_body(write_raw, ce, db, eh, srcr, dstr, zz, *refs):
  if write_raw:
    raw_o, nd_o = refs[0], refs[1]
    scratch = refs[2:]
  else:
    raw_o = None
    nd_o = refs[0]
    scratch = refs[1:]
  acc = scratch[0]
  sets = [_BufSet(*scratch[1 + i * 15:1 + (i + 1) * 15]) for i in range(2)]

  c = lax.axis_index("c")
  s = lax.axis_index("s")
  cN = c * N
  cE = c * E

  pltpu.sync_copy(zz.at[pl.ds(0, NWB)], acc.at[pl.ds(s * NWB, NWB)])

  @pl.when(s == NS - 1)
  def _():
    pltpu.sync_copy(zz.at[pl.ds(0, 16)], acc.at[pl.ds(NS * NWB, 16)])

  base0 = s * EPT

  def issue_idx(b, S):
    o = base0 + b * KB
    pltpu.async_copy(srcr.at[pl.ds(o, KB)], S.is_raw, S.s_idx)
    pltpu.async_copy(dstr.at[pl.ds(o, KB)], S.id_raw, S.s_idx)

  def issue(b, S):
    o = base0 + b * KB
    @pl.when(b >= 2)
    def _():
      if write_raw:
        pltpu.make_async_copy(S.ce_v, raw_o.at[pl.ds(cE + o, KB)],
                              S.s_raw).wait()
      pltpu.make_async_copy(S.cs_v, acc.at[S.idr], S.s_sc).wait()

    pltpu.make_async_copy(srcr.at[pl.ds(o, KB)], S.is_raw, S.s_idx).wait()
    pltpu.make_async_copy(dstr.at[pl.ds(o, KB)], S.id_raw, S.s_idx).wait()

    def cpidx(i, _):
      sl = pl.ds(i * 16, 16)
      d = S.id_raw[sl]
      S.ig[sl] = S.is_raw[sl] + cN
      S.ih[sl] = d + cN
      S.idr[sl] = d
      return 0

    lax.fori_loop(0, KB // 16, cpidx, 0)
    pltpu.async_copy(ce.at[pl.ds(cE + o, KB)], S.ce_v, S.s_ce)
    pltpu.async_copy(db.at[S.ig], S.db_v, S.s_db)
    pltpu.async_copy(eh.at[S.ih], S.eh_v, S.s_eh)

    @pl.when(b + 2 < NBLK)
    def _():
      issue_idx(b + 2, S)

  def process(b, S):
    o = base0 + b * KB
    pltpu.make_async_copy(ce.at[pl.ds(cE + o, KB)], S.ce_v, S.s_ce).wait()
    pltpu.make_async_copy(db.at[S.ig], S.db_v, S.s_db).wait()
    pltpu.make_async_copy(eh.at[S.ih], S.eh_v, S.s_eh).wait()

    def row(r, _):
      for rr in range(2):
        rw = 2 * r + rr
        for j in range(4):
          sl = pl.ds(j * 16, 16)
          sh = pl.ds(HH + j * 16, 16)
          rv = S.ce_v[rw, sl] + S.db_v[rw, sl] + S.eh_v[rw, sl]
          S.ce_v[rw, sl] = rv
          sg = 1.0 / (1.0 + jnp.exp(-rv))
          S.cs_v[rw, sh] = sg
          S.cs_v[rw, sl] = sg * S.db_v[rw, sh]
      return 0

    lax.fori_loop(0, KB // 2, row, 0)

    if write_raw:
      pltpu.async_copy(S.ce_v, raw_o.at[pl.ds(cE + o, KB)], S.s_raw)
    pltpu.async_copy(S.cs_v, acc.at[S.idr], S.s_sc, add=True)

  issue_idx(0, sets[0])
  issue_idx(1, sets[1])
  plsc.subcore_barrier()
  issue(0, sets[0])
  issue(1, sets[1])

  def g_body(g, _):
    b0 = 2 * g
    process(b0, sets[0])
    issue(b0 + 2, sets[0])
    process(b0 + 1, sets[1])
    issue(b0 + 3, sets[1])
    return 0

  lax.fori_loop(0, (NBLK - 3) // 2, g_body, 0)
  process(NBLK - 3, sets[0])
  issue(NBLK - 1, sets[0])
  process(NBLK - 2, sets[1])
  process(NBLK - 1, sets[0])

  for S in sets:
    if write_raw:
      pltpu.make_async_copy(S.ce_v, raw_o.at[pl.ds(cE, KB)], S.s_raw).wait()
    pltpu.make_async_copy(S.cs_v, acc.at[S.idr], S.s_sc).wait()

  plsc.subcore_barrier()

  pltpu.sync_copy(acc.at[pl.ds(s * NWB, NWB)],
                  nd_o.at[pl.ds(cN + s * NWB, NWB)])

  @pl.when(s == NS - 1)
  def _():
    pltpu.sync_copy(acc.at[pl.ds(NS * NWB, 16)],
                    nd_o.at[pl.ds(cN + NS * NWB, 16)])


def _make_edge_kernel(write_raw):
  out_type = [jax.ShapeDtypeStruct((NC * N, H), jnp.float32)]
  if write_raw:
    out_type = [jax.ShapeDtypeStruct((NC * E, HH), jnp.float32)] + out_type
  one_set = [
      pltpu.VMEM((KB,), jnp.int32),
      pltpu.VMEM((KB,), jnp.int32),
      pltpu.VMEM((KB,), jnp.int32),
      pltpu.VMEM((KB,), jnp.int32),
      pltpu.VMEM((KB,), jnp.int32),
      pltpu.VMEM((KB, HH), jnp.float32),
      pltpu.VMEM((KB, H), jnp.float32),
      pltpu.VMEM((KB, HH), jnp.float32),
      pltpu.VMEM((KB, H), jnp.float32),
      pltpu.SemaphoreType.DMA,
      pltpu.SemaphoreType.DMA,
      pltpu.SemaphoreType.DMA,
      pltpu.SemaphoreType.DMA,
      pltpu.SemaphoreType.DMA,
      pltpu.SemaphoreType.DMA,
  ]
  scratch = [
      pltpu.VMEM_SHARED((N, H), jnp.float32),
  ] + one_set + one_set
  return pl.kernel(
      functools.partial(_edge_body, write_raw),
      out_type=out_type,
      mesh=_mesh,
      scratch_types=scratch,
      compiler_params=pltpu.CompilerParams(use_tc_tiling_on_sc=False),
  )


_edge_mid = _make_edge_kernel(True)
_edge_last = _make_edge_kernel(False)


def _bn_relu(x, g, b):
  mean = jnp.mean(x, axis=0, keepdims=True)
  var = jnp.var(x, axis=0, keepdims=True)
  return jax.nn.relu((x - mean) / jnp.sqrt(var + EPS_BN) * g + b)


def _split2(x):
  return jnp.concatenate([x[:, :HH], x[:, HH:]], axis=0)


def kernel(box_feats, text_feats, edge_feat, edge_index, params):
  p = params
  src = edge_index[0].astype(jnp.int32)
  dst = edge_index[1].astype(jnp.int32)
  zz = jnp.zeros((NWB, H), jnp.float32)

  h = jnp.concatenate([box_feats, text_feats], axis=1) @ p['node_enc_w'] + p['node_enc_b']
  e = edge_feat @ p['edge_enc_w'] + p['edge_enc_b']

  n_layers = len(p['layers'])
  for li, lp in enumerate(p['layers']):
    last = li == n_layers - 1
    h_in, e_in = h, e
    Ah = h @ lp['Aw'] + lp['Ab']
    Bh = h @ lp['Bw'] + lp['Bb']
    Ce = e @ lp['Cw'] + lp['Cb']
    Dh = h @ lp['Dw'] + lp['Db']
    Eh = h @ lp['Ew'] + lp['Eb']

    ce2 = _split2(Ce)
    eh2 = _split2(Eh)
    db2 = jnp.concatenate([
        jnp.concatenate([Dh[:, :HH], Bh[:, :HH]], axis=1),
        jnp.concatenate([Dh[:, HH:], Bh[:, HH:]], axis=1),
    ], axis=0)

    if last:
      (nd,) = _edge_last(ce2, db2, eh2, src, dst, zz)
    else:
      raw2, nd = _edge_mid(ce2, db2, eh2, src, dst, zz)

    num = jnp.concatenate([nd[:N, :HH], nd[N:, :HH]], axis=1)
    den = jnp.concatenate([nd[:N, HH:], nd[N:, HH:]], axis=1)

    h_new = Ah + num / (den + EPS)
    h = h_in + _bn_relu(h_new, lp['bnh_g'], lp['bnh_b'])
    if not last:
      raw = jnp.concatenate([raw2[:E], raw2[E:]], axis=1)
      e = e_in + _bn_relu(raw, lp['bne_g'], lp['bne_b'])

  m = jax.nn.relu(h @ p['mlp1_w'] + p['mlp1_b'])
  return m @ p['mlp2_w'] + p['mlp2_b']


# speedup vs baseline: 2.1085x; 1.8366x over previous
import functools

import jax
import jax.numpy as jnp
from jax import lax
from jax.experimental import pallas as pl
from jax.experimental.pallas import tpu as pltpu
from jax.experimental.pallas import tpu_sc as plsc

N = 10000
E = 320000
H = 128
HH = 64
NC = 2
NS = 16
EPT = E // NS
KB = 32
NBLK = EPT // KB
NWB = 624
EPS = 1e-6
EPS_BN = 1e-5

_mesh = plsc.VectorSubcoreMesh(core_axis_name="c", subcore_axis_name="s")


class _BufSet:
  def __init__(self, is_raw, id_raw, ig, ih, idr, ce_v, db_v, eh_v, cs_v,
               s_idx, s_ce, s_db, s_eh, s_raw, s_sc):
    self.is_raw = is_raw
    self.id_raw = id_raw
    self.ig = ig
    self.ih = ih
    self.idr = idr
    self.ce_v = ce_v
    self.db_v = db_v
    self.eh_v = eh_v
    self.cs_v = cs_v
    self.s_idx = s_idx
    self.s_ce = s_ce
    self.s_db = s_db
    self.s_eh = s_eh
    self.s_raw = s_raw
    self.s_sc = s_sc


def _edge_body(write_raw, ce, db, eh, srcr, dstr, zz, *refs):
  if write_raw:
    raw_o, nd_o = refs[0], refs[1]
    scratch = refs[2:]
  else:
    raw_o = None
    nd_o = refs[0]
    scratch = refs[1:]
  acc = scratch[0]
  sets = [_BufSet(*scratch[1 + i * 15:1 + (i + 1) * 15]) for i in range(2)]

  c = lax.axis_index("c")
  s = lax.axis_index("s")
  cN = c * N
  cE = c * E

  pltpu.sync_copy(zz.at[pl.ds(0, NWB)], acc.at[pl.ds(s * NWB, NWB)])

  @pl.when(s == NS - 1)
  def _():
    pltpu.sync_copy(zz.at[pl.ds(0, 16)], acc.at[pl.ds(NS * NWB, 16)])

  base0 = s * EPT

  def issue_idx(b, S):
    o = base0 + b * KB
    pltpu.async_copy(srcr.at[pl.ds(o, KB)], S.is_raw, S.s_idx)
    pltpu.async_copy(dstr.at[pl.ds(o, KB)], S.id_raw, S.s_idx)

  def issue(b, S):
    o = base0 + b * KB
    @pl.when(b >= 2)
    def _():
      if write_raw:
        pltpu.make_async_copy(S.ce_v, raw_o.at[pl.ds(cE + o, KB)],
                              S.s_raw).wait()
      pltpu.make_async_copy(S.cs_v, acc.at[S.idr], S.s_sc).wait()

    pltpu.make_async_copy(srcr.at[pl.ds(o, KB)], S.is_raw, S.s_idx).wait()
    pltpu.make_async_copy(dstr.at[pl.ds(o, KB)], S.id_raw, S.s_idx).wait()

    def cpidx(i, _):
      sl = pl.ds(i * 16, 16)
      d = S.id_raw[sl]
      S.ig[sl] = S.is_raw[sl] + cN
      S.ih[sl] = d + cN
      S.idr[sl] = d
      return 0

    lax.fori_loop(0, KB // 16, cpidx, 0)
    pltpu.async_copy(ce.at[pl.ds(cE + o, KB)], S.ce_v, S.s_ce)
    pltpu.async_copy(db.at[S.ig], S.db_v, S.s_db)
    pltpu.async_copy(eh.at[S.ih], S.eh_v, S.s_eh)

    @pl.when(b + 2 < NBLK)
    def _():
      issue_idx(b + 2, S)

  def process(b, S):
    o = base0 + b * KB
    pltpu.make_async_copy(ce.at[pl.ds(cE + o, KB)], S.ce_v, S.s_ce).wait()
    pltpu.make_async_copy(db.at[S.ig], S.db_v, S.s_db).wait()
    pltpu.make_async_copy(eh.at[S.ih], S.eh_v, S.s_eh).wait()

    @plsc.parallel_loop(0, KB, 1, unroll=4)
    def row(rw):
      for j in range(4):
        sl = pl.ds(j * 16, 16)
        sh = pl.ds(HH + j * 16, 16)
        rv = S.ce_v[rw, sl] + S.db_v[rw, sl] + S.eh_v[rw, sl]
        S.ce_v[rw, sl] = rv
        sg = 1.0 / (1.0 + jnp.exp(-rv))
        S.cs_v[rw, sh] = sg
        S.cs_v[rw, sl] = sg * S.db_v[rw, sh]

    if write_raw:
      pltpu.async_copy(S.ce_v, raw_o.at[pl.ds(cE + o, KB)], S.s_raw)
    pltpu.async_copy(S.cs_v, acc.at[S.idr], S.s_sc, add=True)

  issue_idx(0, sets[0])
  issue_idx(1, sets[1])
  plsc.subcore_barrier()
  issue(0, sets[0])
  issue(1, sets[1])

  def g_body(g, _):
    b0 = 2 * g
    process(b0, sets[0])
    issue(b0 + 2, sets[0])
    process(b0 + 1, sets[1])
    issue(b0 + 3, sets[1])
    return 0

  lax.fori_loop(0, (NBLK - 3) // 2, g_body, 0)
  process(NBLK - 3, sets[0])
  issue(NBLK - 1, sets[0])
  process(NBLK - 2, sets[1])
  process(NBLK - 1, sets[0])

  for S in sets:
    if write_raw:
      pltpu.make_async_copy(S.ce_v, raw_o.at[pl.ds(cE, KB)], S.s_raw).wait()
    pltpu.make_async_copy(S.cs_v, acc.at[S.idr], S.s_sc).wait()

  plsc.subcore_barrier()

  pltpu.sync_copy(acc.at[pl.ds(s * NWB, NWB)],
                  nd_o.at[pl.ds(cN + s * NWB, NWB)])

  @pl.when(s == NS - 1)
  def _():
    pltpu.sync_copy(acc.at[pl.ds(NS * NWB, 16)],
                    nd_o.at[pl.ds(cN + NS * NWB, 16)])


def _make_edge_kernel(write_raw):
  out_type = [jax.ShapeDtypeStruct((NC * N, H), jnp.float32)]
  if write_raw:
    out_type = [jax.ShapeDtypeStruct((NC * E, HH), jnp.float32)] + out_type
  one_set = [
      pltpu.VMEM((KB,), jnp.int32),
      pltpu.VMEM((KB,), jnp.int32),
      pltpu.VMEM((KB,), jnp.int32),
      pltpu.VMEM((KB,), jnp.int32),
      pltpu.VMEM((KB,), jnp.int32),
      pltpu.VMEM((KB, HH), jnp.float32),
      pltpu.VMEM((KB, H), jnp.float32),
      pltpu.VMEM((KB, HH), jnp.float32),
      pltpu.VMEM((KB, H), jnp.float32),
      pltpu.SemaphoreType.DMA,
      pltpu.SemaphoreType.DMA,
      pltpu.SemaphoreType.DMA,
      pltpu.SemaphoreType.DMA,
      pltpu.SemaphoreType.DMA,
      pltpu.SemaphoreType.DMA,
  ]
  scratch = [
      pltpu.VMEM_SHARED((N, H), jnp.float32),
  ] + one_set + one_set
  return pl.kernel(
      functools.partial(_edge_body, write_raw),
      out_type=out_type,
      mesh=_mesh,
      scratch_types=scratch,
      compiler_params=pltpu.CompilerParams(use_tc_tiling_on_sc=False),
  )


_edge_mid = _make_edge_kernel(True)
_edge_last = _make_edge_kernel(False)


def _bn_relu(x, g, b):
  mean = jnp.mean(x, axis=0, keepdims=True)
  var = jnp.var(x, axis=0, keepdims=True)
  return jax.nn.relu((x - mean) / jnp.sqrt(var + EPS_BN) * g + b)


def _split2(x):
  return jnp.concatenate([x[:, :HH], x[:, HH:]], axis=0)


def kernel(box_feats, text_feats, edge_feat, edge_index, params):
  p = params
  src = edge_index[0].astype(jnp.int32)
  dst = edge_index[1].astype(jnp.int32)
  zz = jnp.zeros((NWB, H), jnp.float32)

  h = jnp.concatenate([box_feats, text_feats], axis=1) @ p['node_enc_w'] + p['node_enc_b']
  e = edge_feat @ p['edge_enc_w'] + p['edge_enc_b']

  n_layers = len(p['layers'])
  for li, lp in enumerate(p['layers']):
    last = li == n_layers - 1
    h_in, e_in = h, e
    Ah = h @ lp['Aw'] + lp['Ab']
    Bh = h @ lp['Bw'] + lp['Bb']
    Ce = e @ lp['Cw'] + lp['Cb']
    Dh = h @ lp['Dw'] + lp['Db']
    Eh = h @ lp['Ew'] + lp['Eb']

    ce2 = _split2(Ce)
    eh2 = _split2(Eh)
    db2 = jnp.concatenate([
        jnp.concatenate([Dh[:, :HH], Bh[:, :HH]], axis=1),
        jnp.concatenate([Dh[:, HH:], Bh[:, HH:]], axis=1),
    ], axis=0)

    if last:
      (nd,) = _edge_last(ce2, db2, eh2, src, dst, zz)
    else:
      raw2, nd = _edge_mid(ce2, db2, eh2, src, dst, zz)

    num = jnp.concatenate([nd[:N, :HH], nd[N:, :HH]], axis=1)
    den = jnp.concatenate([nd[:N, HH:], nd[N:, HH:]], axis=1)

    h_new = Ah + num / (den + EPS)
    h = h_in + _bn_relu(h_new, lp['bnh_g'], lp['bnh_b'])
    if not last:
      raw = jnp.concatenate([raw2[:E], raw2[E:]], axis=1)
      e = e_in + _bn_relu(raw, lp['bne_g'], lp['bne_b'])

  m = jax.nn.relu(h @ p['mlp1_w'] + p['mlp1_b'])
  return m @ p['mlp2_w'] + p['mlp2_b']
